# baseline (speedup 1.0000x reference)
import jax
import jax.numpy as jnp
from jax.experimental import pallas as pl
from jax.experimental.pallas import tpu as pltpu

_B, _S, _D, _L = 1, 512, 768, 16
_H = _D // 2


def _hyper_lattice_kernel(x_ref, gate_w_ref, w1_ref, w2_ref, out_w_ref,
                          out_b_ref, ln_g_ref, ln_b_ref, out_ref,
                          acc_ref, idx_ref):
    e = pl.program_id(0)
    x = x_ref[...]

    @pl.when(e == 0)
    def _route():
        logits = jnp.dot(x, gate_w_ref[...].T,
                         preferred_element_type=jnp.float32)
        idx_ref[...] = jnp.argmax(logits, axis=-1, keepdims=True).astype(
            jnp.int32)
        acc_ref[...] = jnp.zeros_like(acc_ref)

    mask = idx_ref[...] == e
    xm = jnp.where(mask, x, 0.0).astype(jnp.bfloat16)
    c1 = jnp.dot(xm, w1_ref[0].astype(jnp.bfloat16),
                 preferred_element_type=jnp.float32)
    c2 = jnp.dot(xm, w2_ref[0].astype(jnp.bfloat16),
                 preferred_element_type=jnp.float32)
    acc_ref[:, :_H] += c1
    acc_ref[:, _H:] += c2

    @pl.when(e == _L - 1)
    def _epilogue():
        out2 = jnp.dot(acc_ref[...].astype(jnp.bfloat16),
                       out_w_ref[...].T.astype(jnp.bfloat16),
                       preferred_element_type=jnp.float32) + out_b_ref[...]
        h = x + out2
        mu = jnp.mean(h, axis=-1, keepdims=True)
        var = jnp.mean((h - mu) ** 2, axis=-1, keepdims=True)
        out_ref[...] = ((h - mu) * jax.lax.rsqrt(var + 1e-5)
                        * ln_g_ref[...] + ln_b_ref[...])


def kernel(x, gate_w, lattice_weights, out_w, out_b, ln_g, ln_b):
    x2 = x.reshape(_S, _D)
    out = pl.pallas_call(
        _hyper_lattice_kernel,
        grid=(_L,),
        in_specs=[
            pl.BlockSpec((_S, _D), lambda e: (0, 0)),
            pl.BlockSpec((_L, _D), lambda e: (0, 0)),
            pl.BlockSpec((1, _D, _H), lambda e: (e, 0, 0)),
            pl.BlockSpec((1, _D, _H), lambda e: (e, 0, 1)),
            pl.BlockSpec((_D, _D), lambda e: (0, 0)),
            pl.BlockSpec((1, _D), lambda e: (0, 0)),
            pl.BlockSpec((1, _D), lambda e: (0, 0)),
            pl.BlockSpec((1, _D), lambda e: (0, 0)),
        ],
        out_specs=pl.BlockSpec((_S, _D), lambda e: (0, 0)),
        out_shape=jax.ShapeDtypeStruct((_S, _D), jnp.float32),
        scratch_shapes=[
            pltpu.VMEM((_S, _D), jnp.float32),
            pltpu.VMEM((_S, 1), jnp.int32),
        ],
    )(x2, gate_w, lattice_weights, lattice_weights, out_w,
      out_b.reshape(1, _D), ln_g.reshape(1, _D), ln_b.reshape(1, _D))
    return out.reshape(_B, _S, _D)


# speedup vs baseline: 1.3258x; 1.3258x over previous
import jax
import jax.numpy as jnp
from jax.experimental import pallas as pl
from jax.experimental.pallas import tpu as pltpu

_B, _S, _D, _L = 1, 512, 768, 16
_EPB = 4
_NSTEP = _L // _EPB


def _hyper_lattice_kernel(x_ref, gate_w_ref, w_ref, out_w_ref, out_b_ref,
                          ln_g_ref, ln_b_ref, out_ref, acc_ref, idx_ref):
    s = pl.program_id(0)
    x = x_ref[...]

    @pl.when(s == 0)
    def _route():
        logits = jnp.dot(x, gate_w_ref[...].T,
                         preferred_element_type=jnp.float32)
        idx_ref[...] = jnp.argmax(logits, axis=-1, keepdims=True).astype(
            jnp.int32)

    c = None
    for j in range(_EPB):
        e = s * _EPB + j
        xm = jnp.where(idx_ref[...] == e, x, 0.0).astype(jnp.bfloat16)
        d = jnp.dot(xm, w_ref[j].astype(jnp.bfloat16),
                    preferred_element_type=jnp.float32)
        c = d if c is None else c + d

    @pl.when(s == 0)
    def _first():
        acc_ref[...] = c

    @pl.when(s > 0)
    def _accum():
        acc_ref[...] += c

    @pl.when(s == _NSTEP - 1)
    def _epilogue():
        out2 = jnp.dot(acc_ref[...].astype(jnp.bfloat16),
                       out_w_ref[...].T.astype(jnp.bfloat16),
                       preferred_element_type=jnp.float32) + out_b_ref[...]
        h = x + out2
        mu = jnp.mean(h, axis=-1, keepdims=True)
        var = jnp.mean((h - mu) ** 2, axis=-1, keepdims=True)
        out_ref[...] = ((h - mu) * jax.lax.rsqrt(var + 1e-5)
                        * ln_g_ref[...] + ln_b_ref[...])


def kernel(x, gate_w, lattice_weights, out_w, out_b, ln_g, ln_b):
    x2 = x.reshape(_S, _D)
    out = pl.pallas_call(
        _hyper_lattice_kernel,
        grid=(_NSTEP,),
        in_specs=[
            pl.BlockSpec((_S, _D), lambda s: (0, 0)),
            pl.BlockSpec((_L, _D), lambda s: (0, 0)),
            pl.BlockSpec((_EPB, _D, _D), lambda s: (s, 0, 0)),
            pl.BlockSpec((_D, _D), lambda s: (0, 0)),
            pl.BlockSpec((1, _D), lambda s: (0, 0)),
            pl.BlockSpec((1, _D), lambda s: (0, 0)),
            pl.BlockSpec((1, _D), lambda s: (0, 0)),
        ],
        out_specs=pl.BlockSpec((_S, _D), lambda s: (0, 0)),
        out_shape=jax.ShapeDtypeStruct((_S, _D), jnp.float32),
        scratch_shapes=[
            pltpu.VMEM((_S, _D), jnp.float32),
            pltpu.VMEM((_S, 1), jnp.int32),
        ],
    )(x2, gate_w, lattice_weights, out_w,
      out_b.reshape(1, _D), ln_g.reshape(1, _D), ln_b.reshape(1, _D))
    return out.reshape(_B, _S, _D)
